# baseline (speedup 1.0000x reference)
import functools

import jax
import jax.numpy as jnp
from jax import lax
from jax.experimental import pallas as pl
from jax.experimental.pallas import tpu as pltpu
from jax.experimental.pallas import tpu_sc as plsc

N = 10000
D = 128
E = 320000
NC = 2
NS = 16
CHUNK = 64
CH_A = 160
CH_B = 320
G = 80
N_ACC = 10112
ROWS_PT = N_ACC // NS
NZ = ROWS_PT // CHUNK
RZ = ROWS_PT - NZ * CHUNK


def _init_zero(src_hbm, buf_v, acc, base):
    pltpu.sync_copy(src_hbm, buf_v)
    for q in range(NZ):
        pltpu.sync_copy(buf_v, acc.at[pl.ds(base + q * CHUNK, CHUNK)])
    pltpu.sync_copy(buf_v.at[pl.ds(0, RZ)],
                    acc.at[pl.ds(base + NZ * CHUNK, RZ)])


def _sc_counts(idx_d, idx_s, z128):
    mesh = plsc.VectorSubcoreMesh(core_axis_name="c", subcore_axis_name="s")

    @functools.partial(
        pl.kernel,
        out_type=[
            jax.ShapeDtypeStruct((NC * N_ACC, D), jnp.float32),
            jax.ShapeDtypeStruct((NC * N_ACC, D), jnp.float32),
        ],
        mesh=mesh,
        scratch_types=[
            pltpu.VMEM((G, CHUNK), jnp.int32),
            pltpu.VMEM((CHUNK, D), jnp.float32),
            pltpu.VMEM_SHARED((N_ACC, D), jnp.float32),
            pltpu.SemaphoreType.DMA,
        ],
    )
    def k(id_hbm, is_hbm, z128_hbm, cd_out, cs_out, is_v, ones_v, acc, csem):
        c = lax.axis_index("c")
        s = lax.axis_index("s")
        blk = c * NS + s
        base = s * ROWS_PT
        ibase = blk * CH_A
        obase = c * N_ACC + base

        for (src_ids, out) in ((id_hbm, cd_out), (is_hbm, cs_out)):
            _init_zero(z128_hbm, ones_v, acc, base)
            def fill(i, _):
                r = i // (D // 16)
                u = i % (D // 16)
                ones_v[r, pl.ds(u * 16, 16)] = jnp.ones((16,), jnp.float32)
                return 0
            lax.fori_loop(0, CHUNK * (D // 16), fill, 0)
            plsc.subcore_barrier()

            for h in range(CH_A // G):
                pltpu.sync_copy(src_ids.at[pl.ds(ibase + h * G, G)], is_v)

                for q in range(G // 20):
                    qb = q * 20

                    def fire(j, _):
                        pltpu.async_copy(
                            ones_v, acc.at[is_v.at[qb + j]], csem, add=True)
                        return 0
                    lax.fori_loop(0, 20, fire, 0)

                    def drain(j, _):
                        pltpu.make_async_copy(
                            ones_v, acc.at[is_v.at[qb + j]], csem).wait()
                        return 0
                    lax.fori_loop(0, 20, drain, 0)

            plsc.subcore_barrier()
            pltpu.sync_copy(acc.at[pl.ds(base, ROWS_PT)],
                            out.at[pl.ds(obase, ROWS_PT)])

    return k(idx_d, idx_s, z128)


def _make_sc(n_chunks):
    mesh = plsc.VectorSubcoreMesh(core_axis_name="c", subcore_axis_name="s")

    @functools.partial(
        pl.kernel,
        out_type=jax.ShapeDtypeStruct((NC * N_ACC, D), jnp.float32),
        mesh=mesh,
        scratch_types=[
            pltpu.VMEM((G, CHUNK), jnp.int32),
            pltpu.VMEM((G, CHUNK), jnp.int32),
            pltpu.VMEM((CHUNK, D), jnp.float32),
            pltpu.VMEM((CHUNK, D), jnp.float32),
            pltpu.VMEM_SHARED((N_ACC, D), jnp.float32),
            pltpu.SemaphoreType.DMA,
            pltpu.SemaphoreType.DMA,
            pltpu.SemaphoreType.DMA,
            pltpu.SemaphoreType.DMA,
        ],
    )
    def k(tab_hbm, ig_hbm, is_hbm, z128_hbm, sum_out,
          ig_v, is_v, rows_a, rows_b, acc, sem_a, sem_b, ssem_a, ssem_b):
        c = lax.axis_index("c")
        s = lax.axis_index("s")
        blk = c * NS + s
        base = s * ROWS_PT
        ibase = blk * n_chunks

        _init_zero(z128_hbm, rows_a, acc, base)
        plsc.subcore_barrier()

        def gath(j, buf, sem):
            pltpu.async_copy(tab_hbm.at[ig_v.at[j]], buf, sem)

        def gwait(j, buf, sem):
            pltpu.make_async_copy(tab_hbm.at[ig_v.at[j]], buf, sem).wait()

        def sstart(j, buf, sem):
            pltpu.async_copy(buf, acc.at[is_v.at[j]], sem, add=True)

        def swait(j, buf, sem):
            pltpu.make_async_copy(buf, acc.at[is_v.at[j]], sem).wait()

        for h in range(n_chunks // G):
            pltpu.sync_copy(ig_hbm.at[pl.ds(ibase + h * G, G)], ig_v)
            pltpu.sync_copy(is_hbm.at[pl.ds(ibase + h * G, G)], is_v)

            gath(0, rows_a, sem_a)
            gath(1, rows_b, sem_b)

            def body(t, _):
                j = 2 * t
                gwait(j, rows_a, sem_a)
                sstart(j, rows_a, ssem_a)
                gwait(j + 1, rows_b, sem_b)
                sstart(j + 1, rows_b, ssem_b)

                @pl.when(t < G // 2 - 1)
                def _():
                    swait(j, rows_a, ssem_a)
                    gath(j + 2, rows_a, sem_a)
                    swait(j + 1, rows_b, ssem_b)
                    gath(j + 3, rows_b, sem_b)
                return 0
            lax.fori_loop(0, G // 2, body, 0)

            swait(G - 2, rows_a, ssem_a)
            swait(G - 1, rows_b, ssem_b)

        plsc.subcore_barrier()
        pltpu.sync_copy(acc.at[pl.ds(base, ROWS_PT)],
                        sum_out.at[pl.ds(c * N_ACC + base, ROWS_PT)])

    return k


_sc_phase_a = _make_sc(CH_A)
_sc_phase_b = _make_sc(CH_B)


BR = 1000


def _mid_body(part_ref, cd_ref, fl_ref, fc_ref, wl_ref, bl_ref,
              wt_ref, wb_ref, bc_ref, tab_ref):
    cnt = cd_ref[:, 0] + cd_ref[:, 1]
    s1 = part_ref[0] + part_ref[1]
    hc = s1 @ wl_ref[...] + cnt[:, None] * bl_ref[...]
    hc = hc / jnp.maximum(cnt, 1.0)[:, None]
    ce = jnp.maximum(hc, 0.0)
    y1 = ce @ wt_ref[...] + fl_ref[...] @ wb_ref[...] + bc_ref[...]
    y2 = jnp.maximum(fc_ref[...] @ wl_ref[...] + bl_ref[...], 0.0)
    tab_ref[0] = y1
    tab_ref[1] = y2


def _tc_mid(part, cnt_d2, feat_literal, feat_clause, W_l2c, b_l2c,
            W_top, W_bot, b_c2l):
    grid = (N // BR,)
    return pl.pallas_call(
        _mid_body,
        grid=grid,
        in_specs=[
            pl.BlockSpec((NC, BR, D), lambda i: (0, i, 0)),
            pl.BlockSpec((BR, NC), lambda i: (i, 0)),
            pl.BlockSpec((BR, D), lambda i: (i, 0)),
            pl.BlockSpec((BR, D), lambda i: (i, 0)),
            pl.BlockSpec((D, D), lambda i: (0, 0)),
            pl.BlockSpec((1, D), lambda i: (0, 0)),
            pl.BlockSpec((D, D), lambda i: (0, 0)),
            pl.BlockSpec((D, D), lambda i: (0, 0)),
            pl.BlockSpec((1, D), lambda i: (0, 0)),
        ],
        out_specs=pl.BlockSpec((NC, BR, D), lambda i: (0, i, 0)),
        out_shape=jax.ShapeDtypeStruct((NC, N, D), jnp.float32),
    )(part, cnt_d2, feat_literal, feat_clause, W_l2c, b_l2c,
      W_top, W_bot, b_c2l)


def _fin_body(tb_ref, cs_ref, out_ref):
    inv = 1.0 / jnp.maximum(cs_ref[:, 0] + cs_ref[:, 1], 1.0)
    out_ref[0] = tb_ref[0] * inv[:, None]
    out_ref[1] = tb_ref[1] * inv[:, None]


def _tc_fin(tb, cnt_s2):
    grid = (N // BR,)
    out = pl.pallas_call(
        _fin_body,
        grid=grid,
        in_specs=[
            pl.BlockSpec((NC, BR, D), lambda i: (0, i, 0)),
            pl.BlockSpec((BR, NC), lambda i: (i, 0)),
        ],
        out_specs=pl.BlockSpec((NC, BR, D), lambda i: (0, i, 0)),
        out_shape=jax.ShapeDtypeStruct((NC, N, D), jnp.float32),
    )(tb, cnt_s2)
    return out[0], out[1]


def _pad_ids(ids, total, fill):
    pad = total - ids.shape[0]
    return jnp.concatenate([ids, jnp.full((pad,), fill, jnp.int32)])


def kernel(feat_literal, feat_clause, edge_index, W_l2c, b_l2c, W_c2l, b_c2l):
    src = edge_index[0]
    dst = edge_index[1]

    ea = NC * NS * CH_A * CHUNK
    ia_g = _pad_ids(src, ea, 0).reshape(NC * NS * CH_A, CHUNK)
    ia_s = _pad_ids(dst, ea, N).reshape(NC * NS * CH_A, CHUNK)
    ia_cs = _pad_ids(src, ea, N).reshape(NC * NS * CH_A, CHUNK)
    eb = NS * CH_B * CHUNK
    g0 = _pad_ids(dst, eb, 0)
    ib_g = jnp.concatenate([g0, g0 + N]).reshape(NC * NS * CH_B, CHUNK)
    s0 = _pad_ids(src, eb, N)
    ib_s = jnp.concatenate([s0, s0]).reshape(NC * NS * CH_B, CHUNK)

    z128 = jnp.zeros((CHUNK, D), jnp.float32)

    cd, cs = _sc_counts(ia_s, ia_cs, z128)
    cnt_d2 = cd.reshape(NC, N_ACC, D)[:, :N, 0].T
    cnt_s2 = cs.reshape(NC, N_ACC, D)[:, :N, 0].T

    part = _sc_phase_a(feat_literal, ia_g, ia_s, z128)
    part = part.reshape(NC, N_ACC, D)

    tables = _tc_mid(
        part[:, :N, :], cnt_d2, feat_literal, feat_clause,
        W_l2c, b_l2c.reshape(1, D), W_c2l[:D], W_c2l[D:],
        b_c2l.reshape(1, D))

    tb = _sc_phase_b(tables.reshape(NC * N, D), ib_g, ib_s, z128)
    tb = tb.reshape(NC, N_ACC, D)

    h_lit, h2 = _tc_fin(tb[:, :N, :], cnt_s2)
    return h_lit, h2


# speedup vs baseline: 1.0676x; 1.0676x over previous
import functools

import jax
import jax.numpy as jnp
from jax import lax
from jax.experimental import pallas as pl
from jax.experimental.pallas import tpu as pltpu
from jax.experimental.pallas import tpu_sc as plsc

N = 10000
D = 128
E = 320000
NC = 2
NS = 16
CHUNK = 64
CH_A = 160
CH_B = 320
G = 80
N_ACC = 10112
ROWS_PT = N_ACC // NS
NZ = ROWS_PT // CHUNK
RZ = ROWS_PT - NZ * CHUNK


def _init_zero(src_hbm, buf_v, acc, base):
    pltpu.sync_copy(src_hbm, buf_v)
    for q in range(NZ):
        pltpu.sync_copy(buf_v, acc.at[pl.ds(base + q * CHUNK, CHUNK)])
    pltpu.sync_copy(buf_v.at[pl.ds(0, RZ)],
                    acc.at[pl.ds(base + NZ * CHUNK, RZ)])


def _sc_counts(idx_d, idx_s, z128):
    mesh = plsc.VectorSubcoreMesh(core_axis_name="c", subcore_axis_name="s")

    @functools.partial(
        pl.kernel,
        out_type=[
            jax.ShapeDtypeStruct((NC * N_ACC, D), jnp.float32),
            jax.ShapeDtypeStruct((NC * N_ACC, D), jnp.float32),
        ],
        mesh=mesh,
        scratch_types=[
            pltpu.VMEM((G, CHUNK), jnp.int32),
            pltpu.VMEM((CHUNK, D), jnp.float32),
            pltpu.VMEM_SHARED((N_ACC, D), jnp.float32),
            pltpu.SemaphoreType.DMA,
        ],
    )
    def k(id_hbm, is_hbm, z128_hbm, cd_out, cs_out, is_v, ones_v, acc, csem):
        c = lax.axis_index("c")
        s = lax.axis_index("s")
        blk = c * NS + s
        base = s * ROWS_PT
        ibase = blk * CH_A
        obase = c * N_ACC + base

        for (src_ids, out) in ((id_hbm, cd_out), (is_hbm, cs_out)):
            _init_zero(z128_hbm, ones_v, acc, base)
            def fill(i, _):
                r = i // (D // 16)
                u = i % (D // 16)
                ones_v[r, pl.ds(u * 16, 16)] = jnp.ones((16,), jnp.float32)
                return 0
            lax.fori_loop(0, CHUNK * (D // 16), fill, 0)
            plsc.subcore_barrier()

            for h in range(CH_A // G):
                pltpu.sync_copy(src_ids.at[pl.ds(ibase + h * G, G)], is_v)

                for q in range(G // 20):
                    qb = q * 20

                    def fire(j, _):
                        pltpu.async_copy(
                            ones_v, acc.at[is_v.at[qb + j]], csem, add=True)
                        return 0
                    lax.fori_loop(0, 20, fire, 0)

                    def drain(j, _):
                        pltpu.make_async_copy(
                            ones_v, acc.at[is_v.at[qb + j]], csem).wait()
                        return 0
                    lax.fori_loop(0, 20, drain, 0)

            plsc.subcore_barrier()
            pltpu.sync_copy(acc.at[pl.ds(base, ROWS_PT)],
                            out.at[pl.ds(obase, ROWS_PT)])

    return k(idx_d, idx_s, z128)


def _make_sc(n_chunks):
    mesh = plsc.VectorSubcoreMesh(core_axis_name="c", subcore_axis_name="s")

    @functools.partial(
        pl.kernel,
        out_type=jax.ShapeDtypeStruct((NC * N_ACC, D), jnp.float32),
        mesh=mesh,
        scratch_types=[
            pltpu.VMEM((G, CHUNK), jnp.int32),
            pltpu.VMEM((G, CHUNK), jnp.int32),
            pltpu.VMEM((CHUNK, D), jnp.float32),
            pltpu.VMEM((CHUNK, D), jnp.float32),
            pltpu.VMEM_SHARED((N_ACC, D), jnp.float32),
            pltpu.SemaphoreType.DMA,
            pltpu.SemaphoreType.DMA,
            pltpu.SemaphoreType.DMA,
            pltpu.SemaphoreType.DMA,
        ],
    )
    def k(tab_hbm, ig_hbm, is_hbm, z128_hbm, sum_out,
          ig_v, is_v, rows_a, rows_b, acc, sem_a, sem_b, ssem_a, ssem_b):
        c = lax.axis_index("c")
        s = lax.axis_index("s")
        blk = c * NS + s
        base = s * ROWS_PT
        ibase = blk * n_chunks

        _init_zero(z128_hbm, rows_a, acc, base)
        plsc.subcore_barrier()

        def gath(j, buf, sem):
            pltpu.async_copy(tab_hbm.at[ig_v.at[j]], buf, sem)

        def gwait(j, buf, sem):
            pltpu.make_async_copy(tab_hbm.at[ig_v.at[j]], buf, sem).wait()

        def sstart(j, buf, sem):
            pltpu.async_copy(buf, acc.at[is_v.at[j]], sem, add=True)

        def swait(j, buf, sem):
            pltpu.make_async_copy(buf, acc.at[is_v.at[j]], sem).wait()

        for h in range(n_chunks // G):
            pltpu.sync_copy(ig_hbm.at[pl.ds(ibase + h * G, G)], ig_v)
            pltpu.sync_copy(is_hbm.at[pl.ds(ibase + h * G, G)], is_v)

            gath(0, rows_a, sem_a)
            gath(1, rows_b, sem_b)

            def body(t, _):
                j = 2 * t
                gwait(j, rows_a, sem_a)
                sstart(j, rows_a, ssem_a)
                swait(j, rows_a, ssem_a)
                gath(j + 2, rows_a, sem_a)
                gwait(j + 1, rows_b, sem_b)
                sstart(j + 1, rows_b, ssem_b)
                swait(j + 1, rows_b, ssem_b)
                gath(j + 3, rows_b, sem_b)
                return 0
            lax.fori_loop(0, G // 2 - 1, body, 0)

            gwait(G - 2, rows_a, sem_a)
            sstart(G - 2, rows_a, ssem_a)
            swait(G - 2, rows_a, ssem_a)
            gwait(G - 1, rows_b, sem_b)
            sstart(G - 1, rows_b, ssem_b)
            swait(G - 1, rows_b, ssem_b)

        plsc.subcore_barrier()
        pltpu.sync_copy(acc.at[pl.ds(base, ROWS_PT)],
                        sum_out.at[pl.ds(c * N_ACC + base, ROWS_PT)])

    return k


_sc_phase_a = _make_sc(CH_A)
_sc_phase_b = _make_sc(CH_B)


BR = 1000


def _mid_body(part_ref, cd_ref, fl_ref, fc_ref, wl_ref, bl_ref,
              wt_ref, wb_ref, bc_ref, tab_ref):
    cnt = cd_ref[:, 0] + cd_ref[:, 1]
    s1 = part_ref[0] + part_ref[1]
    hc = s1 @ wl_ref[...] + cnt[:, None] * bl_ref[...]
    hc = hc / jnp.maximum(cnt, 1.0)[:, None]
    ce = jnp.maximum(hc, 0.0)
    y1 = ce @ wt_ref[...] + fl_ref[...] @ wb_ref[...] + bc_ref[...]
    y2 = jnp.maximum(fc_ref[...] @ wl_ref[...] + bl_ref[...], 0.0)
    tab_ref[0] = y1
    tab_ref[1] = y2


def _tc_mid(part, cnt_d2, feat_literal, feat_clause, W_l2c, b_l2c,
            W_top, W_bot, b_c2l):
    grid = (N // BR,)
    return pl.pallas_call(
        _mid_body,
        grid=grid,
        in_specs=[
            pl.BlockSpec((NC, BR, D), lambda i: (0, i, 0)),
            pl.BlockSpec((BR, NC), lambda i: (i, 0)),
            pl.BlockSpec((BR, D), lambda i: (i, 0)),
            pl.BlockSpec((BR, D), lambda i: (i, 0)),
            pl.BlockSpec((D, D), lambda i: (0, 0)),
            pl.BlockSpec((1, D), lambda i: (0, 0)),
            pl.BlockSpec((D, D), lambda i: (0, 0)),
            pl.BlockSpec((D, D), lambda i: (0, 0)),
            pl.BlockSpec((1, D), lambda i: (0, 0)),
        ],
        out_specs=pl.BlockSpec((NC, BR, D), lambda i: (0, i, 0)),
        out_shape=jax.ShapeDtypeStruct((NC, N, D), jnp.float32),
    )(part, cnt_d2, feat_literal, feat_clause, W_l2c, b_l2c,
      W_top, W_bot, b_c2l)


def _fin_body(tb_ref, cs_ref, out_ref):
    inv = 1.0 / jnp.maximum(cs_ref[:, 0] + cs_ref[:, 1], 1.0)
    out_ref[0] = tb_ref[0] * inv[:, None]
    out_ref[1] = tb_ref[1] * inv[:, None]


def _tc_fin(tb, cnt_s2):
    grid = (N // BR,)
    out = pl.pallas_call(
        _fin_body,
        grid=grid,
        in_specs=[
            pl.BlockSpec((NC, BR, D), lambda i: (0, i, 0)),
            pl.BlockSpec((BR, NC), lambda i: (i, 0)),
        ],
        out_specs=pl.BlockSpec((NC, BR, D), lambda i: (0, i, 0)),
        out_shape=jax.ShapeDtypeStruct((NC, N, D), jnp.float32),
    )(tb, cnt_s2)
    return out[0], out[1]


def _pad_ids(ids, total, fill):
    pad = total - ids.shape[0]
    return jnp.concatenate([ids, jnp.full((pad,), fill, jnp.int32)])


def kernel(feat_literal, feat_clause, edge_index, W_l2c, b_l2c, W_c2l, b_c2l):
    src = edge_index[0]
    dst = edge_index[1]

    ea = NC * NS * CH_A * CHUNK
    ia_g = _pad_ids(src, ea, 0).reshape(NC * NS * CH_A, CHUNK)
    ia_s = _pad_ids(dst, ea, N).reshape(NC * NS * CH_A, CHUNK)
    ia_cs = _pad_ids(src, ea, N).reshape(NC * NS * CH_A, CHUNK)
    eb = NS * CH_B * CHUNK
    g0 = _pad_ids(dst, eb, 0)
    ib_g = jnp.concatenate([g0, g0 + N]).reshape(NC * NS * CH_B, CHUNK)
    s0 = _pad_ids(src, eb, N)
    ib_s = jnp.concatenate([s0, s0]).reshape(NC * NS * CH_B, CHUNK)

    z128 = jnp.zeros((CHUNK, D), jnp.float32)

    cd, cs = _sc_counts(ia_s, ia_cs, z128)
    cnt_d2 = cd.reshape(NC, N_ACC, D)[:, :N, 0].T
    cnt_s2 = cs.reshape(NC, N_ACC, D)[:, :N, 0].T

    part = _sc_phase_a(feat_literal, ia_g, ia_s, z128)
    part = part.reshape(NC, N_ACC, D)

    tables = _tc_mid(
        part[:, :N, :], cnt_d2, feat_literal, feat_clause,
        W_l2c, b_l2c.reshape(1, D), W_c2l[:D], W_c2l[D:],
        b_c2l.reshape(1, D))

    tb = _sc_phase_b(tables.reshape(NC * N, D), ib_g, ib_s, z128)
    tb = tb.reshape(NC, N_ACC, D)

    h_lit, h2 = _tc_fin(tb[:, :N, :], cnt_s2)
    return h_lit, h2


# speedup vs baseline: 1.0995x; 1.0299x over previous
import functools

import jax
import jax.numpy as jnp
from jax import lax
from jax.experimental import pallas as pl
from jax.experimental.pallas import tpu as pltpu
from jax.experimental.pallas import tpu_sc as plsc

N = 10000
D = 128
E = 320000
NC = 2
NS = 16
CHUNK = 64
CH_A = 160
CH_B = 320
G = 80
N_ACC = 10112
ROWS_PT = N_ACC // NS
NZ = ROWS_PT // CHUNK
RZ = ROWS_PT - NZ * CHUNK


def _init_zero(src_hbm, buf_v, acc, base):
    pltpu.sync_copy(src_hbm, buf_v)
    for q in range(NZ):
        pltpu.sync_copy(buf_v, acc.at[pl.ds(base + q * CHUNK, CHUNK)])
    pltpu.sync_copy(buf_v.at[pl.ds(0, RZ)],
                    acc.at[pl.ds(base + NZ * CHUNK, RZ)])


def _sc_counts(idx_d, idx_s, z128):
    mesh = plsc.VectorSubcoreMesh(core_axis_name="c", subcore_axis_name="s")

    @functools.partial(
        pl.kernel,
        out_type=[
            jax.ShapeDtypeStruct((NC * N_ACC, D), jnp.float32),
            jax.ShapeDtypeStruct((NC * N_ACC, D), jnp.float32),
        ],
        mesh=mesh,
        scratch_types=[
            pltpu.VMEM((G, CHUNK), jnp.int32),
            pltpu.VMEM((CHUNK, D), jnp.float32),
            pltpu.VMEM_SHARED((N_ACC, D), jnp.float32),
            pltpu.SemaphoreType.DMA,
        ],
    )
    def k(id_hbm, is_hbm, z128_hbm, cd_out, cs_out, is_v, ones_v, acc, csem):
        c = lax.axis_index("c")
        s = lax.axis_index("s")
        blk = c * NS + s
        base = s * ROWS_PT
        ibase = blk * CH_A
        obase = c * N_ACC + base

        for (src_ids, out) in ((id_hbm, cd_out), (is_hbm, cs_out)):
            _init_zero(z128_hbm, ones_v, acc, base)
            def fill(i, _):
                r = i // (D // 16)
                u = i % (D // 16)
                ones_v[r, pl.ds(u * 16, 16)] = jnp.ones((16,), jnp.float32)
                return 0
            lax.fori_loop(0, CHUNK * (D // 16), fill, 0)
            plsc.subcore_barrier()

            for h in range(CH_A // G):
                pltpu.sync_copy(src_ids.at[pl.ds(ibase + h * G, G)], is_v)

                for q in range(G // 20):
                    qb = q * 20

                    def fire(j, _):
                        pltpu.async_copy(
                            ones_v, acc.at[is_v.at[qb + j]], csem, add=True)
                        return 0
                    lax.fori_loop(0, 20, fire, 0)

                    def drain(j, _):
                        pltpu.make_async_copy(
                            ones_v, acc.at[is_v.at[qb + j]], csem).wait()
                        return 0
                    lax.fori_loop(0, 20, drain, 0)

            plsc.subcore_barrier()
            pltpu.sync_copy(acc.at[pl.ds(base, ROWS_PT)],
                            out.at[pl.ds(obase, ROWS_PT)])

    return k(idx_d, idx_s, z128)


def _make_sc(n_chunks):
    mesh = plsc.VectorSubcoreMesh(core_axis_name="c", subcore_axis_name="s")

    @functools.partial(
        pl.kernel,
        out_type=jax.ShapeDtypeStruct((NC * N_ACC, D), jnp.float32),
        mesh=mesh,
        scratch_types=[
            pltpu.VMEM((G, CHUNK), jnp.int32),
            pltpu.VMEM((G, CHUNK), jnp.int32),
            pltpu.VMEM((CHUNK, D), jnp.float32),
            pltpu.VMEM((CHUNK, D), jnp.float32),
            pltpu.VMEM_SHARED((N_ACC, D), jnp.float32),
            pltpu.SemaphoreType.DMA,
            pltpu.SemaphoreType.DMA,
            pltpu.SemaphoreType.DMA,
            pltpu.SemaphoreType.DMA,
        ],
    )
    def k(tab_hbm, ig_hbm, is_hbm, z128_hbm, sum_out,
          ig_v, is_v, rows_a, rows_b, acc, sem_a, sem_b, ssem_a, ssem_b):
        c = lax.axis_index("c")
        s = lax.axis_index("s")
        base = s * ROWS_PT

        _init_zero(z128_hbm, rows_a, acc, base)
        plsc.subcore_barrier()

        def gath(j, buf, sem):
            pltpu.async_copy(tab_hbm.at[ig_v.at[j]], buf, sem)

        def gwait(j, buf, sem):
            pltpu.make_async_copy(tab_hbm.at[ig_v.at[j]], buf, sem).wait()

        def sstart(j, buf, sem):
            pltpu.async_copy(buf, acc.at[is_v.at[j]], sem, add=True)

        def swait(j, buf, sem):
            pltpu.make_async_copy(buf, acc.at[is_v.at[j]], sem).wait()

        def run_groups(ibase, ngroups):
            for h in range(ngroups):
                pltpu.sync_copy(ig_hbm.at[pl.ds(ibase + h * G, G)], ig_v)
                pltpu.sync_copy(is_hbm.at[pl.ds(ibase + h * G, G)], is_v)

                gath(0, rows_a, sem_a)
                gath(1, rows_b, sem_b)

                def body(t, _):
                    j = 2 * t
                    gwait(j, rows_a, sem_a)
                    sstart(j, rows_a, ssem_a)
                    swait(j, rows_a, ssem_a)
                    gath(j + 2, rows_a, sem_a)
                    gwait(j + 1, rows_b, sem_b)
                    sstart(j + 1, rows_b, ssem_b)
                    swait(j + 1, rows_b, ssem_b)
                    gath(j + 3, rows_b, sem_b)
                    return 0
                lax.fori_loop(0, G // 2 - 1, body, 0)

                gwait(G - 2, rows_a, sem_a)
                sstart(G - 2, rows_a, ssem_a)
                swait(G - 2, rows_a, ssem_a)
                gwait(G - 1, rows_b, sem_b)
                sstart(G - 1, rows_b, ssem_b)
                swait(G - 1, rows_b, ssem_b)

        ch0, ch1 = n_chunks
        if ch0 == ch1:
            run_groups((c * NS + s) * ch0, ch0 // G)
        else:
            @pl.when(c == 0)
            def _():
                run_groups(s * ch0, ch0 // G)

            @pl.when(c == 1)
            def _():
                run_groups(NS * ch0 + s * ch1, ch1 // G)

        plsc.subcore_barrier()
        pltpu.sync_copy(acc.at[pl.ds(base, ROWS_PT)],
                        sum_out.at[pl.ds(c * N_ACC + base, ROWS_PT)])

    return k


CH_A0 = 240
CH_A1 = 2 * CH_A - CH_A0
_sc_phase_a = _make_sc((CH_A0, CH_A1))
_sc_phase_b = _make_sc((CH_B, CH_B))


BR = 1000


def _mid_body(part_ref, cd_ref, fl_ref, fc_ref, wl_ref, bl_ref,
              wt_ref, wb_ref, bc_ref, tab_ref):
    cnt = cd_ref[:, 0] + cd_ref[:, 1]
    s1 = part_ref[0] + part_ref[1]
    hc = s1 @ wl_ref[...] + cnt[:, None] * bl_ref[...]
    hc = hc / jnp.maximum(cnt, 1.0)[:, None]
    ce = jnp.maximum(hc, 0.0)
    y1 = ce @ wt_ref[...] + fl_ref[...] @ wb_ref[...] + bc_ref[...]
    y2 = jnp.maximum(fc_ref[...] @ wl_ref[...] + bl_ref[...], 0.0)
    tab_ref[0] = y1
    tab_ref[1] = y2


def _tc_mid(part, cnt_d2, feat_literal, feat_clause, W_l2c, b_l2c,
            W_top, W_bot, b_c2l):
    grid = (N // BR,)
    return pl.pallas_call(
        _mid_body,
        grid=grid,
        in_specs=[
            pl.BlockSpec((NC, BR, D), lambda i: (0, i, 0)),
            pl.BlockSpec((BR, NC), lambda i: (i, 0)),
            pl.BlockSpec((BR, D), lambda i: (i, 0)),
            pl.BlockSpec((BR, D), lambda i: (i, 0)),
            pl.BlockSpec((D, D), lambda i: (0, 0)),
            pl.BlockSpec((1, D), lambda i: (0, 0)),
            pl.BlockSpec((D, D), lambda i: (0, 0)),
            pl.BlockSpec((D, D), lambda i: (0, 0)),
            pl.BlockSpec((1, D), lambda i: (0, 0)),
        ],
        out_specs=pl.BlockSpec((NC, BR, D), lambda i: (0, i, 0)),
        out_shape=jax.ShapeDtypeStruct((NC, N, D), jnp.float32),
    )(part, cnt_d2, feat_literal, feat_clause, W_l2c, b_l2c,
      W_top, W_bot, b_c2l)


def _fin_body(tb_ref, cs_ref, out_ref):
    inv = 1.0 / jnp.maximum(cs_ref[:, 0] + cs_ref[:, 1], 1.0)
    out_ref[0] = tb_ref[0] * inv[:, None]
    out_ref[1] = tb_ref[1] * inv[:, None]


def _tc_fin(tb, cnt_s2):
    grid = (N // BR,)
    out = pl.pallas_call(
        _fin_body,
        grid=grid,
        in_specs=[
            pl.BlockSpec((NC, BR, D), lambda i: (0, i, 0)),
            pl.BlockSpec((BR, NC), lambda i: (i, 0)),
        ],
        out_specs=pl.BlockSpec((NC, BR, D), lambda i: (0, i, 0)),
        out_shape=jax.ShapeDtypeStruct((NC, N, D), jnp.float32),
    )(tb, cnt_s2)
    return out[0], out[1]


def _pad_ids(ids, total, fill):
    pad = total - ids.shape[0]
    return jnp.concatenate([ids, jnp.full((pad,), fill, jnp.int32)])


def kernel(feat_literal, feat_clause, edge_index, W_l2c, b_l2c, W_c2l, b_c2l):
    src = edge_index[0]
    dst = edge_index[1]

    ea = NC * NS * CH_A * CHUNK
    ia_g = _pad_ids(src, ea, 0).reshape(NC * NS * CH_A, CHUNK)
    ia_s = _pad_ids(dst, ea, N).reshape(NC * NS * CH_A, CHUNK)
    ia_cs = _pad_ids(src, ea, N).reshape(NC * NS * CH_A, CHUNK)
    eb = NS * CH_B * CHUNK
    g0 = _pad_ids(dst, eb, 0)
    ib_g = jnp.concatenate([g0, g0 + N]).reshape(NC * NS * CH_B, CHUNK)
    s0 = _pad_ids(src, eb, N)
    ib_s = jnp.concatenate([s0, s0]).reshape(NC * NS * CH_B, CHUNK)

    z128 = jnp.zeros((CHUNK, D), jnp.float32)

    cd, cs = _sc_counts(ia_s, ia_cs, z128)
    cnt_d2 = cd.reshape(NC, N_ACC, D)[:, :N, 0].T
    cnt_s2 = cs.reshape(NC, N_ACC, D)[:, :N, 0].T

    part = _sc_phase_a(feat_literal, ia_g, ia_s, z128)
    part = part.reshape(NC, N_ACC, D)

    tables = _tc_mid(
        part[:, :N, :], cnt_d2, feat_literal, feat_clause,
        W_l2c, b_l2c.reshape(1, D), W_c2l[:D], W_c2l[D:],
        b_c2l.reshape(1, D))

    tb = _sc_phase_b(tables.reshape(NC * N, D), ib_g, ib_s, z128)
    tb = tb.reshape(NC, N_ACC, D)

    h_lit, h2 = _tc_fin(tb[:, :N, :], cnt_s2)
    return h_lit, h2


# speedup vs baseline: 1.2136x; 1.1038x over previous
import functools

import jax
import jax.numpy as jnp
from jax import lax
from jax.experimental import pallas as pl
from jax.experimental.pallas import tpu as pltpu
from jax.experimental.pallas import tpu_sc as plsc

N = 10000
D = 128
E = 320000
NC = 2
NS = 16
CHUNK = 64
CH_A = 160
CH_B = 320
G = 80
N_ACC = 10112
ROWS_PT = N_ACC // NS
NZ = ROWS_PT // CHUNK
RZ = ROWS_PT - NZ * CHUNK


def _init_zero(src_hbm, buf_v, acc, base):
    pltpu.sync_copy(src_hbm, buf_v)
    for q in range(NZ):
        pltpu.sync_copy(buf_v, acc.at[pl.ds(base + q * CHUNK, CHUNK)])
    pltpu.sync_copy(buf_v.at[pl.ds(0, RZ)],
                    acc.at[pl.ds(base + NZ * CHUNK, RZ)])


def _sc_counts(idx_d, idx_s, z128):
    mesh = plsc.VectorSubcoreMesh(core_axis_name="c", subcore_axis_name="s")

    @functools.partial(
        pl.kernel,
        out_type=[
            jax.ShapeDtypeStruct((NC * N_ACC, D), jnp.float32),
            jax.ShapeDtypeStruct((NC * N_ACC, D), jnp.float32),
        ],
        mesh=mesh,
        scratch_types=[
            pltpu.VMEM((G, CHUNK), jnp.int32),
            pltpu.VMEM((CHUNK, D), jnp.float32),
            pltpu.VMEM_SHARED((N_ACC, D), jnp.float32),
            pltpu.SemaphoreType.DMA,
        ],
    )
    def k(id_hbm, is_hbm, z128_hbm, cd_out, cs_out, is_v, ones_v, acc, csem):
        c = lax.axis_index("c")
        s = lax.axis_index("s")
        blk = c * NS + s
        base = s * ROWS_PT
        ibase = blk * CH_A
        obase = c * N_ACC + base

        for (src_ids, out) in ((id_hbm, cd_out), (is_hbm, cs_out)):
            _init_zero(z128_hbm, ones_v, acc, base)
            def fill(i, _):
                r = i // (D // 16)
                u = i % (D // 16)
                ones_v[r, pl.ds(u * 16, 16)] = jnp.ones((16,), jnp.float32)
                return 0
            lax.fori_loop(0, CHUNK * (D // 16), fill, 0)
            plsc.subcore_barrier()

            for h in range(CH_A // G):
                pltpu.sync_copy(src_ids.at[pl.ds(ibase + h * G, G)], is_v)

                for q in range(G // 20):
                    qb = q * 20

                    def fire(j, _):
                        pltpu.async_copy(
                            ones_v, acc.at[is_v.at[qb + j]], csem, add=True)
                        return 0
                    lax.fori_loop(0, 20, fire, 0)

                    def drain(j, _):
                        pltpu.make_async_copy(
                            ones_v, acc.at[is_v.at[qb + j]], csem).wait()
                        return 0
                    lax.fori_loop(0, 20, drain, 0)

            plsc.subcore_barrier()
            pltpu.sync_copy(acc.at[pl.ds(base, ROWS_PT)],
                            out.at[pl.ds(obase, ROWS_PT)])

    return k(idx_d, idx_s, z128)


def _make_sc(n_chunks):
    mesh = plsc.VectorSubcoreMesh(core_axis_name="c", subcore_axis_name="s")

    @functools.partial(
        pl.kernel,
        out_type=jax.ShapeDtypeStruct((NC * N_ACC, D), jnp.float32),
        mesh=mesh,
        scratch_types=[
            pltpu.VMEM((G, CHUNK), jnp.int32),
            pltpu.VMEM((G, CHUNK), jnp.int32),
            pltpu.VMEM((CHUNK, D), jnp.float32),
            pltpu.VMEM((CHUNK, D), jnp.float32),
            pltpu.VMEM_SHARED((N_ACC, D), jnp.float32),
            pltpu.SemaphoreType.DMA,
            pltpu.SemaphoreType.DMA,
            pltpu.SemaphoreType.DMA,
            pltpu.SemaphoreType.DMA,
        ],
    )
    def k(tab_hbm, ig_hbm, is_hbm, z128_hbm, sum_out,
          ig_v, is_v, rows_a, rows_b, acc, sem_a, sem_b, ssem_a, ssem_b):
        c = lax.axis_index("c")
        s = lax.axis_index("s")
        base = s * ROWS_PT

        _init_zero(z128_hbm, rows_a, acc, base)
        plsc.subcore_barrier()

        def gath(j, buf, sem):
            pltpu.async_copy(tab_hbm.at[ig_v.at[j]], buf, sem)

        def gwait(j, buf, sem):
            pltpu.make_async_copy(tab_hbm.at[ig_v.at[j]], buf, sem).wait()

        def sstart(j, buf, sem):
            pltpu.async_copy(buf, acc.at[is_v.at[j]], sem, add=True)

        def swait(j, buf, sem):
            pltpu.make_async_copy(buf, acc.at[is_v.at[j]], sem).wait()

        def run_groups(ibase, ngroups):
            for h in range(ngroups):
                pltpu.sync_copy(ig_hbm.at[pl.ds(ibase + h * G, G)], ig_v)
                pltpu.sync_copy(is_hbm.at[pl.ds(ibase + h * G, G)], is_v)

                gath(0, rows_a, sem_a)
                gath(1, rows_b, sem_b)

                def body(t, _):
                    j = 2 * t
                    gwait(j, rows_a, sem_a)
                    sstart(j, rows_a, ssem_a)
                    swait(j, rows_a, ssem_a)
                    gath(j + 2, rows_a, sem_a)
                    gwait(j + 1, rows_b, sem_b)
                    sstart(j + 1, rows_b, ssem_b)
                    swait(j + 1, rows_b, ssem_b)
                    gath(j + 3, rows_b, sem_b)
                    return 0
                lax.fori_loop(0, G // 2 - 1, body, 0)

                gwait(G - 2, rows_a, sem_a)
                sstart(G - 2, rows_a, ssem_a)
                swait(G - 2, rows_a, ssem_a)
                gwait(G - 1, rows_b, sem_b)
                sstart(G - 1, rows_b, ssem_b)
                swait(G - 1, rows_b, ssem_b)

        ch0, ch1 = n_chunks
        if ch0 == ch1:
            run_groups((c * NS + s) * ch0, ch0 // G)
        else:
            @pl.when(c == 0)
            def _():
                run_groups(s * ch0, ch0 // G)

            @pl.when(c == 1)
            def _():
                run_groups(NS * ch0 + s * ch1, ch1 // G)

        plsc.subcore_barrier()
        pltpu.sync_copy(acc.at[pl.ds(base, ROWS_PT)],
                        sum_out.at[pl.ds(c * N_ACC + base, ROWS_PT)])

    return k


CH_A0 = 200
CH_A1 = 2 * CH_A - CH_A0
_sc_phase_a = _make_sc((CH_A0, CH_A1))
_sc_phase_b = _make_sc((CH_B, CH_B))


BR = 1000


def _mid_body(part_ref, cd_ref, fl_ref, fc_ref, wl_ref, bl_ref,
              wt_ref, wb_ref, bc_ref, tab_ref):
    cnt = cd_ref[:, 0] + cd_ref[:, 1]
    s1 = part_ref[0] + part_ref[1]
    hc = s1 @ wl_ref[...] + cnt[:, None] * bl_ref[...]
    hc = hc / jnp.maximum(cnt, 1.0)[:, None]
    ce = jnp.maximum(hc, 0.0)
    y1 = ce @ wt_ref[...] + fl_ref[...] @ wb_ref[...] + bc_ref[...]
    y2 = jnp.maximum(fc_ref[...] @ wl_ref[...] + bl_ref[...], 0.0)
    tab_ref[0] = y1
    tab_ref[1] = y2


def _tc_mid(part, cnt_d2, feat_literal, feat_clause, W_l2c, b_l2c,
            W_top, W_bot, b_c2l):
    grid = (N // BR,)
    return pl.pallas_call(
        _mid_body,
        grid=grid,
        in_specs=[
            pl.BlockSpec((NC, BR, D), lambda i: (0, i, 0)),
            pl.BlockSpec((BR, NC), lambda i: (i, 0)),
            pl.BlockSpec((BR, D), lambda i: (i, 0)),
            pl.BlockSpec((BR, D), lambda i: (i, 0)),
            pl.BlockSpec((D, D), lambda i: (0, 0)),
            pl.BlockSpec((1, D), lambda i: (0, 0)),
            pl.BlockSpec((D, D), lambda i: (0, 0)),
            pl.BlockSpec((D, D), lambda i: (0, 0)),
            pl.BlockSpec((1, D), lambda i: (0, 0)),
        ],
        out_specs=pl.BlockSpec((NC, BR, D), lambda i: (0, i, 0)),
        out_shape=jax.ShapeDtypeStruct((NC, N, D), jnp.float32),
    )(part, cnt_d2, feat_literal, feat_clause, W_l2c, b_l2c,
      W_top, W_bot, b_c2l)


def _fin_body(tb_ref, cs_ref, out_ref):
    inv = 1.0 / jnp.maximum(cs_ref[:, 0] + cs_ref[:, 1], 1.0)
    out_ref[0] = tb_ref[0] * inv[:, None]
    out_ref[1] = tb_ref[1] * inv[:, None]


def _tc_fin(tb, cnt_s2):
    grid = (N // BR,)
    out = pl.pallas_call(
        _fin_body,
        grid=grid,
        in_specs=[
            pl.BlockSpec((NC, BR, D), lambda i: (0, i, 0)),
            pl.BlockSpec((BR, NC), lambda i: (i, 0)),
        ],
        out_specs=pl.BlockSpec((NC, BR, D), lambda i: (0, i, 0)),
        out_shape=jax.ShapeDtypeStruct((NC, N, D), jnp.float32),
    )(tb, cnt_s2)
    return out[0], out[1]


def _pad_ids(ids, total, fill):
    pad = total - ids.shape[0]
    return jnp.concatenate([ids, jnp.full((pad,), fill, jnp.int32)])


def kernel(feat_literal, feat_clause, edge_index, W_l2c, b_l2c, W_c2l, b_c2l):
    src = edge_index[0]
    dst = edge_index[1]

    ea = NC * NS * CH_A * CHUNK
    ia_g = _pad_ids(src, ea, 0).reshape(NC * NS * CH_A, CHUNK)
    ia_s = _pad_ids(dst, ea, N).reshape(NC * NS * CH_A, CHUNK)
    ia_cs = _pad_ids(src, ea, N).reshape(NC * NS * CH_A, CHUNK)
    eb = NS * CH_B * CHUNK
    g0 = _pad_ids(dst, eb, 0)
    ib_g = jnp.concatenate([g0, g0 + N]).reshape(NC * NS * CH_B, CHUNK)
    s0 = _pad_ids(src, eb, N)
    ib_s = jnp.concatenate([s0, s0]).reshape(NC * NS * CH_B, CHUNK)

    z128 = jnp.zeros((CHUNK, D), jnp.float32)

    cd, cs = _sc_counts(ia_s, ia_cs, z128)
    cnt_d2 = cd.reshape(NC, N_ACC, D)[:, :N, 0].T
    cnt_s2 = cs.reshape(NC, N_ACC, D)[:, :N, 0].T

    part = _sc_phase_a(feat_literal, ia_g, ia_s, z128)
    part = part.reshape(NC, N_ACC, D)

    tables = _tc_mid(
        part[:, :N, :], cnt_d2, feat_literal, feat_clause,
        W_l2c, b_l2c.reshape(1, D), W_c2l[:D], W_c2l[D:],
        b_c2l.reshape(1, D))

    tb = _sc_phase_b(tables.reshape(NC * N, D), ib_g, ib_s, z128)
    tb = tb.reshape(NC, N_ACC, D)

    h_lit, h2 = _tc_fin(tb[:, :N, :], cnt_s2)
    return h_lit, h2
